# baseline (speedup 1.0000x reference)
import dataclasses

import jax
import jax.numpy as jnp
from jax import lax
from jax.experimental import pallas as pl
from jax.experimental.pallas import tpu as pltpu
from jax.experimental.pallas import tpu_sc as plsc

K = 32
LANES = 16
NUM_WORKERS = 32
BATCH = 16384
CHUNK = BATCH // NUM_WORKERS
GATHER_ROWS = 128
N_STREAMS = CHUNK // GATHER_ROWS
GROUPS = CHUNK // LANES


def _mf_kernel(u_hbm, i_hbm, a_hbm, b_hbm, out_hbm,
               idx_u, idx_i, a_rows, b_rows, out_v, sem):
    wid = lax.axis_index("c") * 16 + lax.axis_index("s")
    row4 = wid * N_STREAMS
    base = wid * CHUNK

    pltpu.sync_copy(u_hbm.at[pl.ds(row4, N_STREAMS)], idx_u)
    pltpu.sync_copy(i_hbm.at[pl.ds(row4, N_STREAMS)], idx_i)

    copies = []
    for j in range(N_STREAMS):
        dst = a_rows.at[pl.ds(j * GATHER_ROWS, GATHER_ROWS)]
        copies.append(pltpu.async_copy(a_hbm.at[idx_u.at[j]], dst, sem))
    for j in range(N_STREAMS):
        dst = b_rows.at[pl.ds(j * GATHER_ROWS, GATHER_ROWS)]
        copies.append(pltpu.async_copy(b_hbm.at[idx_i.at[j]], dst, sem))
    for c in copies:
        c.wait()

    iota = lax.iota(jnp.int32, LANES)

    @pl.loop(0, GROUPS)
    def _(g):
        rows = g * LANES + iota
        acc = jnp.zeros((LANES,), jnp.float32)
        for c in range(K):
            col = jnp.full((LANES,), c, jnp.int32)
            ga = plsc.load_gather(a_rows, [rows, col])
            gb = plsc.load_gather(b_rows, [rows, col])
            acc = acc + ga * gb
        out_v[pl.ds(g * LANES, LANES)] = acc

    pltpu.sync_copy(out_v, out_hbm.at[pl.ds(base, CHUNK)])


@jax.jit
def kernel(user_indices, item_indices, A, B):
    u2d = user_indices.astype(jnp.int32).reshape(128, 128)
    i2d = item_indices.astype(jnp.int32).reshape(128, 128)
    mesh = plsc.VectorSubcoreMesh(core_axis_name="c", subcore_axis_name="s")
    cp = pltpu.CompilerParams(
        needs_layout_passes=False, use_tc_tiling_on_sc=False
    )
    run = pl.kernel(
        _mf_kernel,
        out_type=jax.ShapeDtypeStruct((BATCH,), jnp.float32),
        mesh=mesh,
        scratch_types=[
            pltpu.VMEM((N_STREAMS, GATHER_ROWS), jnp.int32),
            pltpu.VMEM((N_STREAMS, GATHER_ROWS), jnp.int32),
            pltpu.VMEM((CHUNK, K), jnp.float32),
            pltpu.VMEM((CHUNK, K), jnp.float32),
            pltpu.VMEM((CHUNK,), jnp.float32),
            pltpu.SemaphoreType.DMA,
        ],
        compiler_params=cp,
    )
    return run(u2d, i2d, A, B)


# speedup vs baseline: 3.4867x; 3.4867x over previous
import jax
import jax.numpy as jnp
from jax import lax
from jax.experimental import pallas as pl
from jax.experimental.pallas import tpu as pltpu
from jax.experimental.pallas import tpu_sc as plsc

K = 32
LANES = 16
NUM_WORKERS = 32
BATCH = 16384
CHUNK = BATCH // NUM_WORKERS
GROUPS = CHUNK // LANES
PER_STEP = 4
STEPS = CHUNK // PER_STEP


def _mf_kernel(u_hbm, i_hbm, at_hbm, bt_hbm, out_hbm,
               idx_u, idx_i, buf_a, buf_b, a_vals, b_vals, out_v,
               sem0, sem1):
    wid = lax.axis_index("c") * 16 + lax.axis_index("s")
    base = wid * CHUNK

    pltpu.sync_copy(u_hbm.at[pl.ds(base, CHUNK)], idx_u.at[pl.ds(0, CHUNK)])
    pltpu.sync_copy(i_hbm.at[pl.ds(base, CHUNK)], idx_i.at[pl.ds(0, CHUNK)])

    iota = lax.iota(jnp.int32, LANES)

    def aligned_off(u):
        return pl.multiple_of((u >> 7) << 7, 128)

    def fire(us, vs, par, sem):
        for j in range(PER_STEP):
            ou = aligned_off(us[j])
            ov = aligned_off(vs[j])
            pltpu.async_copy(at_hbm.at[:, pl.ds(ou, 128)], buf_a.at[par, j], sem)
            pltpu.async_copy(bt_hbm.at[:, pl.ds(ov, 128)], buf_b.at[par, j], sem)

    def drain(par, sem):
        for j in range(PER_STEP):
            pltpu.make_async_copy(
                at_hbm.at[:, pl.ds(0, 128)], buf_a.at[par, j], sem).wait()
            pltpu.make_async_copy(
                bt_hbm.at[:, pl.ds(0, 128)], buf_b.at[par, j], sem).wait()

    def extract(step, us, vs, par):
        for j in range(PER_STEP):
            i = step * PER_STEP + j
            lane_u = jnp.full((LANES,), us[j] & 127, jnp.int32)
            lane_v = jnp.full((LANES,), vs[j] & 127, jnp.int32)
            cpar = jnp.full((LANES,), par, jnp.int32)
            cj = jnp.full((LANES,), j, jnp.int32)
            for h in range(2):
                krows = iota + h * LANES
                ga = plsc.load_gather(buf_a, [cpar, cj, krows, lane_u])
                gb = plsc.load_gather(buf_b, [cpar, cj, krows, lane_v])
                a_vals[pl.ds(i * K + h * LANES, LANES)] = ga
                b_vals[pl.ds(i * K + h * LANES, LANES)] = gb

    vu0 = idx_u[pl.ds(0, LANES)]
    vi0 = idx_i[pl.ds(0, LANES)]
    fire([vu0[j] for j in range(4)], [vi0[j] for j in range(4)], 0, sem0)
    fire([vu0[4 + j] for j in range(4)], [vi0[4 + j] for j in range(4)], 1, sem1)

    @pl.loop(0, STEPS, step=2)
    def _(s):
        vu = idx_u[pl.ds(s * PER_STEP, LANES)]
        vi = idx_i[pl.ds(s * PER_STEP, LANES)]

        drain(0, sem0)
        extract(s, [vu[j] for j in range(4)], [vi[j] for j in range(4)], 0)

        @pl.when(s + 2 < STEPS)
        def _():
            fire([vu[8 + j] for j in range(4)],
                 [vi[8 + j] for j in range(4)], 0, sem0)

        drain(1, sem1)
        extract(s + 1, [vu[4 + j] for j in range(4)],
                [vi[4 + j] for j in range(4)], 1)

        @pl.when(s + 3 < STEPS)
        def _():
            fire([vu[12 + j] for j in range(4)],
                 [vi[12 + j] for j in range(4)], 1, sem1)

    @pl.loop(0, GROUPS)
    def _(g):
        rows = (g * LANES + iota) * K
        acc = jnp.zeros((LANES,), jnp.float32)
        for c in range(K):
            ga = plsc.load_gather(a_vals, [rows + c])
            gb = plsc.load_gather(b_vals, [rows + c])
            acc = acc + ga * gb
        out_v[pl.ds(g * LANES, LANES)] = acc

    pltpu.sync_copy(out_v, out_hbm.at[pl.ds(base, CHUNK)])


@jax.jit
def kernel(user_indices, item_indices, A, B):
    u1d = user_indices.astype(jnp.int32).reshape(BATCH)
    i1d = item_indices.astype(jnp.int32).reshape(BATCH)
    mesh = plsc.VectorSubcoreMesh(core_axis_name="c", subcore_axis_name="s")
    cp = pltpu.CompilerParams(
        use_tc_tiling_on_sc=True, needs_layout_passes=False
    )
    run = pl.kernel(
        _mf_kernel,
        out_type=jax.ShapeDtypeStruct((BATCH,), jnp.float32),
        mesh=mesh,
        scratch_types=[
            pltpu.VMEM((CHUNK + LANES,), jnp.int32),
            pltpu.VMEM((CHUNK + LANES,), jnp.int32),
            pltpu.VMEM((2, PER_STEP, K, 128), jnp.float32),
            pltpu.VMEM((2, PER_STEP, K, 128), jnp.float32),
            pltpu.VMEM((CHUNK * K,), jnp.float32),
            pltpu.VMEM((CHUNK * K,), jnp.float32),
            pltpu.VMEM((CHUNK,), jnp.float32),
            pltpu.SemaphoreType.DMA,
            pltpu.SemaphoreType.DMA,
        ],
        compiler_params=cp,
    )
    return run(u1d, i1d, A.T, B.T)
